# baseline (speedup 1.0000x reference)
import jax
import jax.numpy as jnp
from jax import lax
from jax.experimental import pallas as pl
from jax.experimental.pallas import tpu as pltpu
from jax.experimental.pallas import tpu_sc as plsc

B, T, D = 4, 4096, 2048
NC, NS = 2, 16
NW = NC * NS
T_SC = 512
T_TC = T - T_SC
CT = T_SC // NW
CHUNK = CT * D


def _add_body(x_ref, t_ref, o_ref):
    o_ref[...] = x_ref[...] + t_ref[...]


def _sc_body(x_hbm, t_hbm, o_hbm, tbuf, xbuf):
    wid = lax.axis_index("s") * NC + lax.axis_index("c")
    src = (T_TC * D) + wid * CHUNK
    dst = wid * CHUNK
    pltpu.sync_copy(t_hbm.at[pl.ds(src, CHUNK)], tbuf)
    for b in range(B):
        pltpu.sync_copy(x_hbm.at[b, pl.ds(src, CHUNK)], xbuf)

        def add8(i, carry):
            for k in range(8):
                off = (i * 8 + k) * 16
                plsc.addupdate(xbuf.at[pl.ds(off, 16)], tbuf[pl.ds(off, 16)])
            return carry

        lax.fori_loop(0, CHUNK // 128, add8, 0)
        pltpu.sync_copy(xbuf, o_hbm.at[b, pl.ds(dst, CHUNK)])


def kernel(x, table):
    BS = 512
    o1 = pl.pallas_call(
        _add_body,
        grid=(T_TC // BS, B),
        in_specs=[
            pl.BlockSpec((1, BS, D), lambda s, b: (b, s, 0)),
            pl.BlockSpec((BS, D), lambda s, b: (s, 0)),
        ],
        out_specs=pl.BlockSpec((1, BS, D), lambda s, b: (b, s, 0)),
        out_shape=jax.ShapeDtypeStruct(x.shape, x.dtype),
    )(x, table)

    sc = pl.kernel(
        _sc_body,
        mesh=plsc.VectorSubcoreMesh(core_axis_name="c", subcore_axis_name="s"),
        out_type=jax.ShapeDtypeStruct((B, T_SC * D), jnp.float32),
        scratch_types=[
            pltpu.VMEM((CHUNK,), jnp.float32),
            pltpu.VMEM((CHUNK,), jnp.float32),
        ],
    )(x.reshape(B, T * D), table.reshape(T * D))

    return lax.dynamic_update_slice(o1, sc.reshape(B, T_SC, D), (0, T_TC, 0))
